# baseline (speedup 1.0000x reference)
import jax
import jax.numpy as jnp
from jax import lax
from jax.experimental import pallas as pl
from jax.experimental.pallas import tpu as pltpu
from jax.experimental.pallas import tpu_sc as plsc

NUM_QUANT = 8
NUM_EMB = 1034
EMB_DIM = 1024
B = 4
T = 4096

NC = 2
NS = 16
NW = NC * NS
LANES = 16

TOKENS = B * T
TOK_PER_W = TOKENS // NW
C = 4
STEPS = TOK_PER_W // C
ROWS_PER_STEP = C * NUM_QUANT
HCHUNKS = EMB_DIM // LANES
NBUF = 3


def _sc_body(ftab_hbm, idx_hbm, out_hbm, idx_v, rows_v, acc_v,
             sem0, sem1, sem2, osem0, osem1, osem2):
    wid = lax.axis_index("s") * NC + lax.axis_index("c")
    base = wid * TOK_PER_W

    for q in range(NUM_QUANT):
        pltpu.sync_copy(idx_hbm.at[q, wid], idx_v.at[q])

    sems = (sem0, sem1, sem2)
    osems = (osem0, osem1, osem2)

    def gather_start(g, b):
        for q in range(NUM_QUANT):
            pltpu.make_async_copy(
                ftab_hbm.at[idx_v.at[q, pl.ds(g * C, C)]],
                rows_v.at[b, q], sems[b]
            ).start()

    def gather_wait(g, b):
        for q in range(NUM_QUANT):
            pltpu.make_async_copy(
                ftab_hbm.at[idx_v.at[q, pl.ds(g * C, C)]],
                rows_v.at[b, q], sems[b]
            ).wait()

    def out_copy(g, b):
        return pltpu.make_async_copy(
            acc_v.at[b], out_hbm.at[pl.ds(base + g * C, C)], osems[b])

    for b in range(NBUF):
        gather_start(b, b)

    def step(g, b):
        gather_wait(g, b)
        @pl.when(g >= NBUF)
        def _():
            out_copy(g - NBUF, b).wait()
        @plsc.parallel_loop(0, EMB_DIM, step=LANES, unroll=2)
        def _(off):
            col = pl.ds(off, LANES)
            for c in range(C):
                acc = rows_v[b, 0, c, col]
                for q in range(1, NUM_QUANT):
                    acc = acc + rows_v[b, q, c, col]
                acc_v[b, c, col] = acc
        out_copy(g, b).start()
        @pl.when(g + NBUF < STEPS)
        def _():
            gather_start(g + NBUF, b)

    def ring(p, _):
        for b in range(NBUF):
            step(p * NBUF + b, b)
        return 0

    lax.fori_loop(0, STEPS // NBUF, ring, 0)
    for r in range(STEPS - STEPS % NBUF, STEPS):
        step(r, r % NBUF)
    for r in range(STEPS - NBUF, STEPS):
        out_copy(r, r % NBUF).wait()


@jax.jit
def _multi_embedding_sum(flat_tables, idx):
    mesh = plsc.VectorSubcoreMesh(core_axis_name="c", subcore_axis_name="s")
    scratch = [
        pltpu.VMEM((NUM_QUANT, TOK_PER_W), jnp.int32),
        pltpu.VMEM((NBUF, NUM_QUANT, C, EMB_DIM), jnp.float32),
        pltpu.VMEM((NBUF, C, EMB_DIM), jnp.float32),
        pltpu.SemaphoreType.DMA,
        pltpu.SemaphoreType.DMA,
        pltpu.SemaphoreType.DMA,
        pltpu.SemaphoreType.DMA,
        pltpu.SemaphoreType.DMA,
        pltpu.SemaphoreType.DMA,
    ]
    run = pl.kernel(
        _sc_body,
        out_type=jax.ShapeDtypeStruct((TOKENS, EMB_DIM), jnp.float32),
        mesh=mesh,
        scratch_types=scratch,
    )
    return run(flat_tables, idx)


def kernel(input_ids, tables):
    flat_tables = tables.reshape(NUM_QUANT * NUM_EMB, EMB_DIM)
    offs = (jnp.arange(NUM_QUANT, dtype=jnp.int32) * NUM_EMB)[:, None, None]
    idx = input_ids.reshape(NUM_QUANT, NW, TOK_PER_W) + offs
    out = _multi_embedding_sum(flat_tables, idx)
    return out.reshape(B, T, EMB_DIM)


# speedup vs baseline: 2.1513x; 2.1513x over previous
import jax
import jax.numpy as jnp
from jax import lax
from jax.experimental import pallas as pl
from jax.experimental.pallas import tpu as pltpu
from jax.experimental.pallas import tpu_sc as plsc

NUM_QUANT = 8
NUM_EMB = 1034
EMB_DIM = 1024
B = 4
T = 4096

NC = 2
NS = 16
NW = NC * NS
LANES = 16

TOKENS = B * T
TOK_PER_W = TOKENS // NW
C = 2
STEPS = TOK_PER_W // C
ROWS_PER_STEP = C * NUM_QUANT
HCHUNKS = EMB_DIM // LANES
NBUF = 6


def _sc_body(ftab_hbm, idx_hbm, out_hbm, idx_v, rows_v, acc_v,
             sem0, sem1, sem2, sem3, sem4, sem5,
             osem0, osem1, osem2, osem3, osem4, osem5):
    wid = lax.axis_index("s") * NC + lax.axis_index("c")
    base = wid * TOK_PER_W

    for q in range(NUM_QUANT):
        pltpu.sync_copy(idx_hbm.at[q, wid], idx_v.at[q])

    sems = (sem0, sem1, sem2, sem3, sem4, sem5)
    osems = (osem0, osem1, osem2, osem3, osem4, osem5)

    def gather_start(g, b):
        for q in range(NUM_QUANT):
            pltpu.make_async_copy(
                ftab_hbm.at[idx_v.at[q, pl.ds(g * C, C)]],
                rows_v.at[b, q], sems[b]
            ).start()

    def gather_wait(g, b):
        for q in range(NUM_QUANT):
            pltpu.make_async_copy(
                ftab_hbm.at[idx_v.at[q, pl.ds(g * C, C)]],
                rows_v.at[b, q], sems[b]
            ).wait()

    def out_copy(g, b):
        return pltpu.make_async_copy(
            acc_v.at[b], out_hbm.at[pl.ds(base + g * C, C)], osems[b])

    for b in range(NBUF):
        gather_start(b, b)

    def step(g, b):
        gather_wait(g, b)
        @pl.when(g >= NBUF)
        def _():
            out_copy(g - NBUF, b).wait()
        @plsc.parallel_loop(0, EMB_DIM, step=LANES, unroll=4)
        def _(off):
            col = pl.ds(off, LANES)
            for c in range(C):
                acc = rows_v[b, 0, c, col]
                for q in range(1, NUM_QUANT):
                    acc = acc + rows_v[b, q, c, col]
                acc_v[b, c, col] = acc
        out_copy(g, b).start()
        @pl.when(g + NBUF < STEPS)
        def _():
            gather_start(g + NBUF, b)

    def ring(p, _):
        for b in range(NBUF):
            step(p * NBUF + b, b)
        return 0

    lax.fori_loop(0, STEPS // NBUF, ring, 0)
    for r in range(STEPS - STEPS % NBUF, STEPS):
        step(r, r % NBUF)
    for r in range(STEPS - NBUF, STEPS):
        out_copy(r, r % NBUF).wait()


@jax.jit
def _multi_embedding_sum(flat_tables, idx):
    mesh = plsc.VectorSubcoreMesh(core_axis_name="c", subcore_axis_name="s")
    scratch = [
        pltpu.VMEM((NUM_QUANT, TOK_PER_W), jnp.int32),
        pltpu.VMEM((NBUF, NUM_QUANT, C, EMB_DIM), jnp.float32),
        pltpu.VMEM((NBUF, C, EMB_DIM), jnp.float32),
        pltpu.SemaphoreType.DMA,
        pltpu.SemaphoreType.DMA,
        pltpu.SemaphoreType.DMA,
        pltpu.SemaphoreType.DMA,
        pltpu.SemaphoreType.DMA,
        pltpu.SemaphoreType.DMA,
        pltpu.SemaphoreType.DMA,
        pltpu.SemaphoreType.DMA,
        pltpu.SemaphoreType.DMA,
        pltpu.SemaphoreType.DMA,
        pltpu.SemaphoreType.DMA,
        pltpu.SemaphoreType.DMA,
    ]
    run = pl.kernel(
        _sc_body,
        out_type=jax.ShapeDtypeStruct((TOKENS, EMB_DIM), jnp.float32),
        mesh=mesh,
        scratch_types=scratch,
    )
    return run(flat_tables, idx)


def kernel(input_ids, tables):
    flat_tables = tables.reshape(NUM_QUANT * NUM_EMB, EMB_DIM)
    offs = (jnp.arange(NUM_QUANT, dtype=jnp.int32) * NUM_EMB)[:, None, None]
    idx = input_ids.reshape(NUM_QUANT, NW, TOK_PER_W) + offs
    out = _multi_embedding_sum(flat_tables, idx)
    return out.reshape(B, T, EMB_DIM)


# speedup vs baseline: 2.1663x; 1.0070x over previous
import jax
import jax.numpy as jnp
from jax import lax
from jax.experimental import pallas as pl
from jax.experimental.pallas import tpu as pltpu
from jax.experimental.pallas import tpu_sc as plsc

NUM_QUANT = 8
NUM_EMB = 1034
EMB_DIM = 1024
B = 4
T = 4096

NC = 2
NS = 16
NW = NC * NS
LANES = 16

TOKENS = B * T
TOK_PER_W = TOKENS // NW
C = 4
STEPS = TOK_PER_W // C
ROWS_PER_STEP = C * NUM_QUANT
HCHUNKS = EMB_DIM // LANES
NBUF = 3


def _sc_body(ftab_hbm, idx_hbm, out_hbm, idx_v, rows_v, acc_v,
             sem0, sem1, sem2, osem0, osem1, osem2):
    wid = lax.axis_index("s") * NC + lax.axis_index("c")
    base = wid * TOK_PER_W

    for q in range(NUM_QUANT):
        pltpu.sync_copy(idx_hbm.at[q, wid], idx_v.at[q])

    sems = (sem0, sem1, sem2)
    osems = (osem0, osem1, osem2)

    def gather_start(g, b):
        for q in range(NUM_QUANT):
            pltpu.make_async_copy(
                ftab_hbm.at[idx_v.at[q, pl.ds(g * C, C)]],
                rows_v.at[b, q], sems[b]
            ).start()

    def gather_wait(g, b):
        for q in range(NUM_QUANT):
            pltpu.make_async_copy(
                ftab_hbm.at[idx_v.at[q, pl.ds(g * C, C)]],
                rows_v.at[b, q], sems[b]
            ).wait()

    def out_copy(g, b):
        return pltpu.make_async_copy(
            acc_v.at[b], out_hbm.at[pl.ds(base + g * C, C)], osems[b])

    for b in range(NBUF):
        gather_start(b, b)

    def step(g, b):
        gather_wait(g, b)
        @pl.when(g >= NBUF)
        def _():
            out_copy(g - NBUF, b).wait()
        @plsc.parallel_loop(0, EMB_DIM, step=LANES, unroll=4)
        def _(off):
            col = pl.ds(off, LANES)
            for c in range(C):
                acc = rows_v[b, 0, c, col]
                for q in range(1, NUM_QUANT):
                    acc = acc + rows_v[b, q, c, col]
                acc_v[b, c, col] = acc
        out_copy(g, b).start()
        @pl.when(g + NBUF < STEPS)
        def _():
            gather_start(g + NBUF, b)

    def ring(p, _):
        for b in range(NBUF):
            step(p * NBUF + b, b)
        return 0

    lax.fori_loop(0, STEPS // NBUF, ring, 0)
    for r in range(STEPS - STEPS % NBUF, STEPS):
        step(r, r % NBUF)
    for r in range(STEPS - NBUF, STEPS):
        out_copy(r, r % NBUF).wait()


@jax.jit
def _multi_embedding_sum(flat_tables, idx):
    mesh = plsc.VectorSubcoreMesh(core_axis_name="c", subcore_axis_name="s")
    scratch = [
        pltpu.VMEM((NUM_QUANT, TOK_PER_W), jnp.int32),
        pltpu.VMEM((NBUF, NUM_QUANT, C, EMB_DIM), jnp.float32),
        pltpu.VMEM((NBUF, C, EMB_DIM), jnp.float32),
        pltpu.SemaphoreType.DMA,
        pltpu.SemaphoreType.DMA,
        pltpu.SemaphoreType.DMA,
        pltpu.SemaphoreType.DMA,
        pltpu.SemaphoreType.DMA,
        pltpu.SemaphoreType.DMA,
    ]
    run = pl.kernel(
        _sc_body,
        out_type=jax.ShapeDtypeStruct((TOKENS, EMB_DIM), jnp.float32),
        mesh=mesh,
        scratch_types=scratch,
    )
    return run(flat_tables, idx)


def kernel(input_ids, tables):
    flat_tables = tables.reshape(NUM_QUANT * NUM_EMB, EMB_DIM)
    offs = (jnp.arange(NUM_QUANT, dtype=jnp.int32) * NUM_EMB)[:, None, None]
    idx = input_ids.reshape(NUM_QUANT, NW, TOK_PER_W) + offs
    out = _multi_embedding_sum(flat_tables, idx)
    return out.reshape(B, T, EMB_DIM)
